# Initial kernel scaffold; baseline (speedup 1.0000x reference)
import jax
import jax.numpy as jnp
from jax.experimental import pallas as pl


def kernel(x, edge_index, edge_weight, Wr, br, Wu, bu, Wc, bc, W_lin, b_lin):
    raise NotImplementedError("write your pallas kernel here")


# speedup vs baseline: 5.1281x; 5.1281x over previous
import functools

import jax
import jax.numpy as jnp
from jax import lax
from jax.experimental import pallas as pl
from jax.experimental.pallas import tpu as pltpu
from jax.experimental.pallas import tpu_sc as plsc

N = 10000
E = 320000
F = 128
NP = 10240
NC = 2
NS = 16
RPT = NP // NS
CA = 80
EA = E // (NC * NS)
EP = E // NS
BT = 1024

_mesh = plsc.VectorSubcoreMesh(core_axis_name="c", subcore_axis_name="s")
_f32 = jnp.float32
_sc_params = pltpu.CompilerParams(needs_layout_passes=False,
                                  use_tc_tiling_on_sc=False)


def _deg_body(dst_h, w_h, z_h, out_h, dbuf, wbuf, wrows, acc):
    c = lax.axis_index("c")
    s = lax.axis_index("s")
    pltpu.sync_copy(z_h, acc.at[pl.ds(s * RPT, RPT)])
    plsc.subcore_barrier()
    base0 = c * (E // NC) + s * EA

    def chunk(i, carry):
        base = base0 + i * CA
        pltpu.sync_copy(dst_h.at[pl.ds(base, CA)], dbuf)
        pltpu.sync_copy(w_h.at[pl.ds(base, CA)], wbuf)
        for g in range(CA // 16):
            w16 = wbuf[pl.ds(g * 16, 16)]
            for e in range(16):
                wrows[g * 16 + e, pl.ds(0, 16)] = jnp.full((16,), w16[e], _f32)
        pltpu.sync_copy(wrows, acc.at[dbuf], add=True)
        return carry

    lax.fori_loop(0, EA // CA, chunk, 0)
    plsc.subcore_barrier()
    pltpu.sync_copy(acc.at[pl.ds(s * RPT, RPT)],
                    out_h.at[c, pl.ds(s * RPT, RPT)])


def _sc_degree(dst, w, z16):
    return pl.kernel(
        _deg_body,
        out_type=jax.ShapeDtypeStruct((NC, NP, 16), _f32),
        mesh=_mesh,
        compiler_params=_sc_params,
        scratch_types=[
            pltpu.VMEM((CA,), jnp.int32),
            pltpu.VMEM((CA,), _f32),
            pltpu.VMEM((CA, 16), _f32),
            pltpu.VMEM_SHARED((NP, 16), _f32),
        ],
    )(dst, w, z16)


def _dinv_body(d_ref, o_ref):
    d = d_ref[0, :, 0:1] + d_ref[1, :, 0:1]
    o_ref[...] = jnp.where(d > 0, lax.rsqrt(d), 0.0)


def _tc_dinv(deg2):
    out = pl.pallas_call(
        _dinv_body,
        out_shape=jax.ShapeDtypeStruct((NP, 1), _f32),
    )(deg2)
    return out.reshape(NP)


def _a_body(src_h, dst_h, w_h, dinv_h, a_out, sbuf, dbuf, wbuf, abuf, dvbuf):
    c = lax.axis_index("c")
    s = lax.axis_index("s")
    pltpu.sync_copy(dinv_h, dvbuf)
    base0 = (c * NS + s) * EA

    def chunk(i, carry):
        base = base0 + i * CA
        pltpu.sync_copy(src_h.at[pl.ds(base, CA)], sbuf)
        pltpu.sync_copy(dst_h.at[pl.ds(base, CA)], dbuf)
        pltpu.sync_copy(w_h.at[pl.ds(base, CA)], wbuf)
        for g in range(CA // 16):
            s16 = sbuf[pl.ds(g * 16, 16)]
            d16 = dbuf[pl.ds(g * 16, 16)]
            w16 = wbuf[pl.ds(g * 16, 16)]
            a16 = (plsc.load_gather(dvbuf, [s16]) * w16
                   * plsc.load_gather(dvbuf, [d16]))
            abuf[pl.ds(g * 16, 16)] = a16
        pltpu.sync_copy(abuf, a_out.at[pl.ds(base, CA)])
        return carry

    lax.fori_loop(0, EA // CA, chunk, 0)


def _sc_edge_norm(src, dst, w, dinv):
    return pl.kernel(
        _a_body,
        out_type=jax.ShapeDtypeStruct((E,), _f32),
        mesh=_mesh,
        compiler_params=_sc_params,
        scratch_types=[
            pltpu.VMEM((CA,), jnp.int32),
            pltpu.VMEM((CA,), jnp.int32),
            pltpu.VMEM((CA,), _f32),
            pltpu.VMEM((CA,), _f32),
            pltpu.VMEM((NP,), _f32),
        ],
    )(src, dst, w, dinv)


def _prop_body(src_h, dst_h, a_h, m_h, z_h, out_h,
               sbuf, dbuf, abuf, rows, acc, sem):
    c = lax.axis_index("c")
    s = lax.axis_index("s")
    pltpu.sync_copy(z_h, acc.at[pl.ds(s * RPT, RPT)])
    plsc.subcore_barrier()
    base0 = s * EP
    coff = jnp.full((16,), c * NP, jnp.int32)

    def chunk(i, carry):
        base = base0 + i * CA
        pltpu.sync_copy(src_h.at[pl.ds(base, CA)], sbuf)
        pltpu.sync_copy(dst_h.at[pl.ds(base, CA)], dbuf)
        pltpu.sync_copy(a_h.at[pl.ds(base, CA)], abuf)
        for g in range(CA // 16):
            sbuf[pl.ds(g * 16, 16)] = sbuf[pl.ds(g * 16, 16)] + coff
        pltpu.async_copy(m_h.at[sbuf], rows, sem).wait()
        for g in range(CA // 16):
            a16 = abuf[pl.ds(g * 16, 16)]
            for e in range(16):
                r = g * 16 + e
                av = jnp.full((16,), a16[e], _f32)
                for j in range(4):
                    rows[r, pl.ds(j * 16, 16)] = rows[r, pl.ds(j * 16, 16)] * av
        pltpu.sync_copy(rows, acc.at[dbuf], add=True)
        return carry

    lax.fori_loop(0, EP // CA, chunk, 0)
    plsc.subcore_barrier()
    pltpu.sync_copy(acc.at[pl.ds(s * RPT, RPT)],
                    out_h.at[c, pl.ds(s * RPT, RPT)])


def _sc_prop(src, dst, a, m_flat, z64):
    return pl.kernel(
        _prop_body,
        out_type=jax.ShapeDtypeStruct((NC, NP, 64), _f32),
        mesh=_mesh,
        compiler_params=_sc_params,
        scratch_types=[
            pltpu.VMEM((CA,), jnp.int32),
            pltpu.VMEM((CA,), jnp.int32),
            pltpu.VMEM((CA,), _f32),
            pltpu.VMEM((CA, 64), _f32),
            pltpu.VMEM_SHARED((NP, 64), _f32),
            pltpu.SemaphoreType.DMA,
        ],
    )(src, dst, a, m_flat, z64)


def _cat(ref):
    return jnp.concatenate([ref[0], ref[1]], axis=1)


def _dot(a, b):
    return jnp.dot(a, b, preferred_element_type=_f32)


def _cell1_body(x_ref, px_ref, p2x_ref, w_ref, b_ref, h2f_ref, h2s_ref):
    x = x_ref[...]
    px = _cat(px_ref)
    p2x = _cat(p2x_ref)
    sacc = _dot(x, w_ref[0]) + _dot(px, w_ref[1]) + _dot(2.0 * p2x - x, w_ref[2])
    sacc = sacc + b_ref[...]
    u = jax.nn.sigmoid(sacc[:, :F])
    cc = jnp.tanh(sacc[:, F:])
    h2 = (1.0 - u) * cc
    h2f_ref[...] = h2
    h2s_ref[0] = h2[:, :64]
    h2s_ref[1] = h2[:, 64:]


def _tc_cell1(xp, px, p2x, wcat, bcat):
    nb = NP // BT
    return pl.pallas_call(
        _cell1_body,
        grid=(nb,),
        in_specs=[
            pl.BlockSpec((BT, F), lambda i: (i, 0)),
            pl.BlockSpec((NC, BT, 64), lambda i: (0, i, 0)),
            pl.BlockSpec((NC, BT, 64), lambda i: (0, i, 0)),
            pl.BlockSpec((3, F, 2 * F), lambda i: (0, 0, 0)),
            pl.BlockSpec((1, 2 * F), lambda i: (0, 0)),
        ],
        out_specs=[
            pl.BlockSpec((BT, F), lambda i: (i, 0)),
            pl.BlockSpec((NC, BT, 64), lambda i: (0, i, 0)),
        ],
        out_shape=[
            jax.ShapeDtypeStruct((NP, F), _f32),
            jax.ShapeDtypeStruct((NC, NP, 64), _f32),
        ],
    )(xp, px, p2x, wcat, bcat)


def _cell2a_body(x_ref, px_ref, p2x_ref, h2_ref, ph2_ref, p2h2_ref,
                 wf_ref, wh_ref, b_ref, u2_ref, gs_ref):
    x = x_ref[...]
    h2 = h2_ref[...]
    px = _cat(px_ref)
    p2x = _cat(p2x_ref)
    ph2 = _cat(ph2_ref)
    p2h2 = _cat(p2h2_ref)
    sacc = (_dot(x, wf_ref[0]) + _dot(px, wf_ref[1])
            + _dot(2.0 * p2x - x, wf_ref[2])
            + _dot(h2, wh_ref[0]) + _dot(ph2, wh_ref[1])
            + _dot(2.0 * p2h2 - h2, wh_ref[2]))
    sacc = sacc + b_ref[...]
    r = jax.nn.sigmoid(sacc[:, :F])
    u2 = jax.nn.sigmoid(sacc[:, F:])
    g = r * h2
    u2_ref[...] = u2
    gs_ref[0] = g[:, :64]
    gs_ref[1] = g[:, 64:]


def _tc_cell2a(xp, px, p2x, h2f, ph2, p2h2, wf, wh, bcat):
    nb = NP // BT
    return pl.pallas_call(
        _cell2a_body,
        grid=(nb,),
        in_specs=[
            pl.BlockSpec((BT, F), lambda i: (i, 0)),
            pl.BlockSpec((NC, BT, 64), lambda i: (0, i, 0)),
            pl.BlockSpec((NC, BT, 64), lambda i: (0, i, 0)),
            pl.BlockSpec((BT, F), lambda i: (i, 0)),
            pl.BlockSpec((NC, BT, 64), lambda i: (0, i, 0)),
            pl.BlockSpec((NC, BT, 64), lambda i: (0, i, 0)),
            pl.BlockSpec((3, F, 2 * F), lambda i: (0, 0, 0)),
            pl.BlockSpec((3, F, 2 * F), lambda i: (0, 0, 0)),
            pl.BlockSpec((1, 2 * F), lambda i: (0, 0)),
        ],
        out_specs=[
            pl.BlockSpec((BT, F), lambda i: (i, 0)),
            pl.BlockSpec((NC, BT, 64), lambda i: (0, i, 0)),
        ],
        out_shape=[
            jax.ShapeDtypeStruct((NP, F), _f32),
            jax.ShapeDtypeStruct((NC, NP, 64), _f32),
        ],
    )(xp, px, p2x, h2f, ph2, p2h2, wf, wh, bcat)


def _cell2b_body(x_ref, px_ref, p2x_ref, gs_ref, pg_ref, p2g_ref,
                 h2_ref, u2_ref, wcf_ref, wch_ref, bc_ref, wl_ref, bl_ref,
                 y_ref):
    x = x_ref[...]
    h2 = h2_ref[...]
    u2 = u2_ref[...]
    px = _cat(px_ref)
    p2x = _cat(p2x_ref)
    g = _cat(gs_ref)
    pg = _cat(pg_ref)
    p2g = _cat(p2g_ref)
    sacc = (_dot(x, wcf_ref[0]) + _dot(px, wcf_ref[1])
            + _dot(2.0 * p2x - x, wcf_ref[2])
            + _dot(g, wch_ref[0]) + _dot(pg, wch_ref[1])
            + _dot(2.0 * p2g - g, wch_ref[2]))
    sacc = sacc + bc_ref[...]
    c2 = jnp.tanh(sacc)
    h3 = u2 * h2 + (1.0 - u2) * c2
    yv = (jnp.sum(h2 * wl_ref[0, :F][None, :], axis=1)
          + jnp.sum(h3 * wl_ref[0, F:][None, :], axis=1)
          + bl_ref[0, 0])
    y_ref[...] = jnp.maximum(yv, 0.0)


def _tc_cell2b(xp, px, p2x, gs, pg, p2g, h2f, u2f, wcf, wch, bc2, wl2, bl2):
    nb = NP // BT
    return pl.pallas_call(
        _cell2b_body,
        grid=(nb,),
        in_specs=[
            pl.BlockSpec((BT, F), lambda i: (i, 0)),
            pl.BlockSpec((NC, BT, 64), lambda i: (0, i, 0)),
            pl.BlockSpec((NC, BT, 64), lambda i: (0, i, 0)),
            pl.BlockSpec((NC, BT, 64), lambda i: (0, i, 0)),
            pl.BlockSpec((NC, BT, 64), lambda i: (0, i, 0)),
            pl.BlockSpec((NC, BT, 64), lambda i: (0, i, 0)),
            pl.BlockSpec((BT, F), lambda i: (i, 0)),
            pl.BlockSpec((BT, F), lambda i: (i, 0)),
            pl.BlockSpec((3, F, F), lambda i: (0, 0, 0)),
            pl.BlockSpec((3, F, F), lambda i: (0, 0, 0)),
            pl.BlockSpec((1, F), lambda i: (0, 0)),
            pl.BlockSpec((1, 2 * F), lambda i: (0, 0)),
            pl.BlockSpec((1, 1), lambda i: (0, 0)),
        ],
        out_specs=pl.BlockSpec((BT,), lambda i: (i,)),
        out_shape=jax.ShapeDtypeStruct((NP,), _f32),
    )(xp, px, p2x, gs, pg, p2g, h2f, u2f, wcf, wch, bc2, wl2, bl2)


def _split_planes(m):
    return jnp.concatenate([m[:, :64], m[:, 64:]], axis=0)


def kernel(x, edge_index, edge_weight, Wr, br, Wu, bu, Wc, bc, W_lin, b_lin):
    src = edge_index[0].astype(jnp.int32)
    dst = edge_index[1].astype(jnp.int32)
    w = edge_weight.astype(_f32)

    xp = jnp.pad(x, ((0, NP - N), (0, 0)))
    xsc = _split_planes(xp)

    wcat1 = jnp.concatenate([Wu[:, :F, :], Wc[:, :F, :]], axis=2)
    bcat1 = jnp.concatenate([bu, bc]).reshape(1, 2 * F)
    wf2 = jnp.concatenate([Wr[:, :F, :], Wu[:, :F, :]], axis=2)
    wh2 = jnp.concatenate([Wr[:, F:, :], Wu[:, F:, :]], axis=2)
    bcat2 = jnp.concatenate([br, bu]).reshape(1, 2 * F)
    wcf = Wc[:, :F, :]
    wch = Wc[:, F:, :]
    bc2 = bc.reshape(1, F)
    wl2 = W_lin.reshape(1, 2 * F)
    bl2 = b_lin.reshape(1, 1)

    z16 = jnp.zeros((RPT, 16), _f32)
    z64 = jnp.zeros((RPT, 64), _f32)

    deg2 = _sc_degree(dst, w, z16)
    dinv = _tc_dinv(deg2)
    a = _sc_edge_norm(src, dst, w, dinv)

    px = _sc_prop(src, dst, a, xsc, z64)
    p2x = _sc_prop(src, dst, a, px.reshape(NC * NP, 64), z64)

    h2f, h2s = _tc_cell1(xp, px, p2x, wcat1, bcat1)

    ph2 = _sc_prop(src, dst, a, h2s.reshape(NC * NP, 64), z64)
    p2h2 = _sc_prop(src, dst, a, ph2.reshape(NC * NP, 64), z64)

    u2f, gs = _tc_cell2a(xp, px, p2x, h2f, ph2, p2h2, wf2, wh2, bcat2)

    pg = _sc_prop(src, dst, a, gs.reshape(NC * NP, 64), z64)
    p2g = _sc_prop(src, dst, a, pg.reshape(NC * NP, 64), z64)

    y = _tc_cell2b(xp, px, p2x, gs, pg, p2g, h2f, u2f, wcf, wch, bc2, wl2, bl2)

    return (y[:N].reshape(N, 1), a)


# speedup vs baseline: 13.8993x; 2.7104x over previous
import functools

import jax
import jax.numpy as jnp
from jax import lax
from jax.experimental import pallas as pl
from jax.experimental.pallas import tpu as pltpu
from jax.experimental.pallas import tpu_sc as plsc

N = 10000
E = 320000
F = 128
NP = 10240
NC = 2
NS = 16
RPT = NP // NS
CA = 80
EA = E // (NC * NS)
EP = E // NS
BT = 1024

_mesh = plsc.VectorSubcoreMesh(core_axis_name="c", subcore_axis_name="s")
_f32 = jnp.float32
_sc_params = pltpu.CompilerParams(needs_layout_passes=False,
                                  use_tc_tiling_on_sc=False)


def _deg_body(dst_h, w_h, z_h, out_h, dbuf, wbuf, wrows, acc):
    c = lax.axis_index("c")
    s = lax.axis_index("s")
    pltpu.sync_copy(z_h, acc.at[pl.ds(s * RPT, RPT)])
    plsc.subcore_barrier()
    base0 = c * (E // NC) + s * EA

    def chunk(i, carry):
        base = base0 + i * CA
        pltpu.sync_copy(dst_h.at[pl.ds(base, CA)], dbuf)
        pltpu.sync_copy(w_h.at[pl.ds(base, CA)], wbuf)
        for g in range(CA // 16):
            w16 = wbuf[pl.ds(g * 16, 16)]
            for e in range(16):
                wrows[g * 16 + e, pl.ds(0, 16)] = jnp.full((16,), w16[e], _f32)
        pltpu.sync_copy(wrows, acc.at[dbuf], add=True)
        return carry

    lax.fori_loop(0, EA // CA, chunk, 0)
    plsc.subcore_barrier()
    pltpu.sync_copy(acc.at[pl.ds(s * RPT, RPT)],
                    out_h.at[c, pl.ds(s * RPT, RPT)])


def _sc_degree(dst, w, z16):
    return pl.kernel(
        _deg_body,
        out_type=jax.ShapeDtypeStruct((NC, NP, 16), _f32),
        mesh=_mesh,
        compiler_params=_sc_params,
        scratch_types=[
            pltpu.VMEM((CA,), jnp.int32),
            pltpu.VMEM((CA,), _f32),
            pltpu.VMEM((CA, 16), _f32),
            pltpu.VMEM_SHARED((NP, 16), _f32),
        ],
    )(dst, w, z16)


def _dinv_body(d_ref, o_ref):
    d = d_ref[0, :, 0:1] + d_ref[1, :, 0:1]
    o_ref[...] = jnp.where(d > 0, lax.rsqrt(d), 0.0)


def _tc_dinv(deg2):
    out = pl.pallas_call(
        _dinv_body,
        out_shape=jax.ShapeDtypeStruct((NP, 1), _f32),
    )(deg2)
    return out.reshape(NP)


def _a_body(src_h, dst_h, w_h, dinv_h, a_out, sbuf, dbuf, wbuf, abuf, dvbuf):
    c = lax.axis_index("c")
    s = lax.axis_index("s")
    pltpu.sync_copy(dinv_h, dvbuf)
    base0 = (c * NS + s) * EA

    def chunk(i, carry):
        base = base0 + i * CA
        pltpu.sync_copy(src_h.at[pl.ds(base, CA)], sbuf)
        pltpu.sync_copy(dst_h.at[pl.ds(base, CA)], dbuf)
        pltpu.sync_copy(w_h.at[pl.ds(base, CA)], wbuf)
        for g in range(CA // 16):
            s16 = sbuf[pl.ds(g * 16, 16)]
            d16 = dbuf[pl.ds(g * 16, 16)]
            w16 = wbuf[pl.ds(g * 16, 16)]
            a16 = (plsc.load_gather(dvbuf, [s16]) * w16
                   * plsc.load_gather(dvbuf, [d16]))
            abuf[pl.ds(g * 16, 16)] = a16
        pltpu.sync_copy(abuf, a_out.at[pl.ds(base, CA)])
        return carry

    lax.fori_loop(0, EA // CA, chunk, 0)


def _sc_edge_norm(src, dst, w, dinv):
    return pl.kernel(
        _a_body,
        out_type=jax.ShapeDtypeStruct((E,), _f32),
        mesh=_mesh,
        compiler_params=_sc_params,
        scratch_types=[
            pltpu.VMEM((CA,), jnp.int32),
            pltpu.VMEM((CA,), jnp.int32),
            pltpu.VMEM((CA,), _f32),
            pltpu.VMEM((CA,), _f32),
            pltpu.VMEM((NP,), _f32),
        ],
    )(src, dst, w, dinv)


NCH = EP // CA


def _scale_rows(rows_b, abigf, abase):
    for g in range(CA // 16):
        a16 = abigf[pl.ds(abase + g * 16, 16)]
        for e in range(16):
            r = g * 16 + e
            av = jnp.full((16,), a16[e], _f32)
            for j in range(4):
                rows_b[r, pl.ds(j * 16, 16)] = rows_b[r, pl.ds(j * 16, 16)] * av


def _prop_body(src_h, dst2_h, a_h, m_h, z_h, out_h,
               sbigf, dbig, abigf, rows0, rows1, acc, sem0, sem1):
    c = lax.axis_index("c")
    s = lax.axis_index("s")
    pltpu.sync_copy(z_h, acc.at[pl.ds(s * RPT, RPT)])
    pltpu.sync_copy(src_h.at[pl.ds(s * EP, EP)], sbigf)
    pltpu.sync_copy(a_h.at[pl.ds(s * EP, EP)], abigf)
    pltpu.sync_copy(dst2_h.at[pl.ds(s * NCH, NCH)], dbig)
    coff = jnp.full((16,), c * NP, jnp.int32)

    def offs(i, carry):
        sbigf[pl.ds(i * 16, 16)] = sbigf[pl.ds(i * 16, 16)] + coff
        return carry

    lax.fori_loop(0, EP // 16, offs, 0)
    plsc.subcore_barrier()

    pltpu.async_copy(m_h.at[sbigf.at[pl.ds(0, CA)]], rows0, sem0)
    pltpu.async_copy(m_h.at[sbigf.at[pl.ds(CA, CA)]], rows1, sem1)

    def pair(i2, carry):
        for b in range(2):
            idx = i2 * 2 + b
            rows_b = rows0 if b == 0 else rows1
            sem_b = sem0 if b == 0 else sem1
            pltpu.make_async_copy(
                m_h.at[sbigf.at[pl.ds(0, CA)]], rows_b, sem_b).wait()
            _scale_rows(rows_b, abigf, idx * CA)
            pltpu.sync_copy(rows_b, acc.at[dbig.at[idx]], add=True)

            @pl.when(idx + 2 < NCH)
            def _():
                pltpu.async_copy(
                    m_h.at[sbigf.at[pl.ds((idx + 2) * CA, CA)]], rows_b, sem_b)
        return carry

    lax.fori_loop(0, NCH // 2, pair, 0)
    plsc.subcore_barrier()
    pltpu.sync_copy(acc.at[pl.ds(s * RPT, RPT)],
                    out_h.at[c, pl.ds(s * RPT, RPT)])


def _sc_prop(src, dst2, a, m_flat, z64):
    return pl.kernel(
        _prop_body,
        out_type=jax.ShapeDtypeStruct((NC, NP, 64), _f32),
        mesh=_mesh,
        compiler_params=_sc_params,
        scratch_types=[
            pltpu.VMEM((EP,), jnp.int32),
            pltpu.VMEM((NCH, CA), jnp.int32),
            pltpu.VMEM((EP,), _f32),
            pltpu.VMEM((CA, 64), _f32),
            pltpu.VMEM((CA, 64), _f32),
            pltpu.VMEM_SHARED((NP, 64), _f32),
            pltpu.SemaphoreType.DMA,
            pltpu.SemaphoreType.DMA,
        ],
    )(src, dst2, a, m_flat, z64)


def _cat(ref):
    return jnp.concatenate([ref[0], ref[1]], axis=1)


def _dot(a, b):
    return jnp.dot(a, b, preferred_element_type=_f32)


def _cell1_body(x_ref, px_ref, p2x_ref, w_ref, b_ref, h2f_ref, h2s_ref):
    x = x_ref[...]
    px = _cat(px_ref)
    p2x = _cat(p2x_ref)
    sacc = _dot(x, w_ref[0]) + _dot(px, w_ref[1]) + _dot(2.0 * p2x - x, w_ref[2])
    sacc = sacc + b_ref[...]
    u = jax.nn.sigmoid(sacc[:, :F])
    cc = jnp.tanh(sacc[:, F:])
    h2 = (1.0 - u) * cc
    h2f_ref[...] = h2
    h2s_ref[0] = h2[:, :64]
    h2s_ref[1] = h2[:, 64:]


def _tc_cell1(xp, px, p2x, wcat, bcat):
    nb = NP // BT
    return pl.pallas_call(
        _cell1_body,
        grid=(nb,),
        in_specs=[
            pl.BlockSpec((BT, F), lambda i: (i, 0)),
            pl.BlockSpec((NC, BT, 64), lambda i: (0, i, 0)),
            pl.BlockSpec((NC, BT, 64), lambda i: (0, i, 0)),
            pl.BlockSpec((3, F, 2 * F), lambda i: (0, 0, 0)),
            pl.BlockSpec((1, 2 * F), lambda i: (0, 0)),
        ],
        out_specs=[
            pl.BlockSpec((BT, F), lambda i: (i, 0)),
            pl.BlockSpec((NC, BT, 64), lambda i: (0, i, 0)),
        ],
        out_shape=[
            jax.ShapeDtypeStruct((NP, F), _f32),
            jax.ShapeDtypeStruct((NC, NP, 64), _f32),
        ],
    )(xp, px, p2x, wcat, bcat)


def _cell2a_body(x_ref, px_ref, p2x_ref, h2_ref, ph2_ref, p2h2_ref,
                 wf_ref, wh_ref, b_ref, u2_ref, gs_ref):
    x = x_ref[...]
    h2 = h2_ref[...]
    px = _cat(px_ref)
    p2x = _cat(p2x_ref)
    ph2 = _cat(ph2_ref)
    p2h2 = _cat(p2h2_ref)
    sacc = (_dot(x, wf_ref[0]) + _dot(px, wf_ref[1])
            + _dot(2.0 * p2x - x, wf_ref[2])
            + _dot(h2, wh_ref[0]) + _dot(ph2, wh_ref[1])
            + _dot(2.0 * p2h2 - h2, wh_ref[2]))
    sacc = sacc + b_ref[...]
    r = jax.nn.sigmoid(sacc[:, :F])
    u2 = jax.nn.sigmoid(sacc[:, F:])
    g = r * h2
    u2_ref[...] = u2
    gs_ref[0] = g[:, :64]
    gs_ref[1] = g[:, 64:]


def _tc_cell2a(xp, px, p2x, h2f, ph2, p2h2, wf, wh, bcat):
    nb = NP // BT
    return pl.pallas_call(
        _cell2a_body,
        grid=(nb,),
        in_specs=[
            pl.BlockSpec((BT, F), lambda i: (i, 0)),
            pl.BlockSpec((NC, BT, 64), lambda i: (0, i, 0)),
            pl.BlockSpec((NC, BT, 64), lambda i: (0, i, 0)),
            pl.BlockSpec((BT, F), lambda i: (i, 0)),
            pl.BlockSpec((NC, BT, 64), lambda i: (0, i, 0)),
            pl.BlockSpec((NC, BT, 64), lambda i: (0, i, 0)),
            pl.BlockSpec((3, F, 2 * F), lambda i: (0, 0, 0)),
            pl.BlockSpec((3, F, 2 * F), lambda i: (0, 0, 0)),
            pl.BlockSpec((1, 2 * F), lambda i: (0, 0)),
        ],
        out_specs=[
            pl.BlockSpec((BT, F), lambda i: (i, 0)),
            pl.BlockSpec((NC, BT, 64), lambda i: (0, i, 0)),
        ],
        out_shape=[
            jax.ShapeDtypeStruct((NP, F), _f32),
            jax.ShapeDtypeStruct((NC, NP, 64), _f32),
        ],
    )(xp, px, p2x, h2f, ph2, p2h2, wf, wh, bcat)


def _cell2b_body(x_ref, px_ref, p2x_ref, gs_ref, pg_ref, p2g_ref,
                 h2_ref, u2_ref, wcf_ref, wch_ref, bc_ref, wl_ref, bl_ref,
                 y_ref):
    x = x_ref[...]
    h2 = h2_ref[...]
    u2 = u2_ref[...]
    px = _cat(px_ref)
    p2x = _cat(p2x_ref)
    g = _cat(gs_ref)
    pg = _cat(pg_ref)
    p2g = _cat(p2g_ref)
    sacc = (_dot(x, wcf_ref[0]) + _dot(px, wcf_ref[1])
            + _dot(2.0 * p2x - x, wcf_ref[2])
            + _dot(g, wch_ref[0]) + _dot(pg, wch_ref[1])
            + _dot(2.0 * p2g - g, wch_ref[2]))
    sacc = sacc + bc_ref[...]
    c2 = jnp.tanh(sacc)
    h3 = u2 * h2 + (1.0 - u2) * c2
    yv = (jnp.sum(h2 * wl_ref[0, :F][None, :], axis=1)
          + jnp.sum(h3 * wl_ref[0, F:][None, :], axis=1)
          + bl_ref[0, 0])
    y_ref[...] = jnp.maximum(yv, 0.0)


def _tc_cell2b(xp, px, p2x, gs, pg, p2g, h2f, u2f, wcf, wch, bc2, wl2, bl2):
    nb = NP // BT
    return pl.pallas_call(
        _cell2b_body,
        grid=(nb,),
        in_specs=[
            pl.BlockSpec((BT, F), lambda i: (i, 0)),
            pl.BlockSpec((NC, BT, 64), lambda i: (0, i, 0)),
            pl.BlockSpec((NC, BT, 64), lambda i: (0, i, 0)),
            pl.BlockSpec((NC, BT, 64), lambda i: (0, i, 0)),
            pl.BlockSpec((NC, BT, 64), lambda i: (0, i, 0)),
            pl.BlockSpec((NC, BT, 64), lambda i: (0, i, 0)),
            pl.BlockSpec((BT, F), lambda i: (i, 0)),
            pl.BlockSpec((BT, F), lambda i: (i, 0)),
            pl.BlockSpec((3, F, F), lambda i: (0, 0, 0)),
            pl.BlockSpec((3, F, F), lambda i: (0, 0, 0)),
            pl.BlockSpec((1, F), lambda i: (0, 0)),
            pl.BlockSpec((1, 2 * F), lambda i: (0, 0)),
            pl.BlockSpec((1, 1), lambda i: (0, 0)),
        ],
        out_specs=pl.BlockSpec((BT,), lambda i: (i,)),
        out_shape=jax.ShapeDtypeStruct((NP,), _f32),
    )(xp, px, p2x, gs, pg, p2g, h2f, u2f, wcf, wch, bc2, wl2, bl2)


def _split_planes(m):
    return jnp.concatenate([m[:, :64], m[:, 64:]], axis=0)


def kernel(x, edge_index, edge_weight, Wr, br, Wu, bu, Wc, bc, W_lin, b_lin):
    src = edge_index[0].astype(jnp.int32)
    dst = edge_index[1].astype(jnp.int32)
    w = edge_weight.astype(_f32)

    xp = jnp.pad(x, ((0, NP - N), (0, 0)))
    xsc = _split_planes(xp)

    wcat1 = jnp.concatenate([Wu[:, :F, :], Wc[:, :F, :]], axis=2)
    bcat1 = jnp.concatenate([bu, bc]).reshape(1, 2 * F)
    wf2 = jnp.concatenate([Wr[:, :F, :], Wu[:, :F, :]], axis=2)
    wh2 = jnp.concatenate([Wr[:, F:, :], Wu[:, F:, :]], axis=2)
    bcat2 = jnp.concatenate([br, bu]).reshape(1, 2 * F)
    wcf = Wc[:, :F, :]
    wch = Wc[:, F:, :]
    bc2 = bc.reshape(1, F)
    wl2 = W_lin.reshape(1, 2 * F)
    bl2 = b_lin.reshape(1, 1)

    z16 = jnp.zeros((RPT, 16), _f32)
    z64 = jnp.zeros((RPT, 64), _f32)
    dst2 = dst.reshape(E // CA, CA)

    deg2 = _sc_degree(dst, w, z16)
    dinv = _tc_dinv(deg2)
    a = _sc_edge_norm(src, dst, w, dinv)

    px = _sc_prop(src, dst2, a, xsc, z64)
    p2x = _sc_prop(src, dst2, a, px.reshape(NC * NP, 64), z64)

    h2f, h2s = _tc_cell1(xp, px, p2x, wcat1, bcat1)

    ph2 = _sc_prop(src, dst2, a, h2s.reshape(NC * NP, 64), z64)
    p2h2 = _sc_prop(src, dst2, a, ph2.reshape(NC * NP, 64), z64)

    u2f, gs = _tc_cell2a(xp, px, p2x, h2f, ph2, p2h2, wf2, wh2, bcat2)

    pg = _sc_prop(src, dst2, a, gs.reshape(NC * NP, 64), z64)
    p2g = _sc_prop(src, dst2, a, pg.reshape(NC * NP, 64), z64)

    y = _tc_cell2b(xp, px, p2x, gs, pg, p2g, h2f, u2f, wcf, wch, bc2, wl2, bl2)

    return (y[:N].reshape(N, 1), a)
